# baseline (speedup 1.0000x reference)
import functools

import jax
import jax.numpy as jnp
from jax import lax
from jax.experimental import pallas as pl
from jax.experimental.pallas import tpu as pltpu
from jax.experimental.pallas import tpu_sc as plsc

B, H, W, C = 4, 112, 112, 192
oH, oW = 2 * H, 2 * W
HW = H * W
oHW = oH * oW
CB = 16
NCB = C // CB
NC, NS = 2, 16
NTASK = B * NCB
TPC = NTASK // NC
EPT = HW * CB // NS
NCHUNK = EPT // 128
ACC = oHW * CB
OPT = ACC // NS
ZCH = 6272

_mesh = plsc.VectorSubcoreMesh(core_axis_name="c", subcore_axis_name="s")


@functools.partial(
    pl.kernel,
    mesh=_mesh,
    out_type=jax.ShapeDtypeStruct((NTASK, ACC), jnp.float32),
    compiler_params=pltpu.CompilerParams(use_tc_tiling_on_sc=False),
    scratch_types=[
        pltpu.VMEM((EPT,), jnp.float32),
        pltpu.VMEM((EPT,), jnp.int32),
        pltpu.VMEM((NCHUNK, 128), jnp.float32),
        pltpu.VMEM((NCHUNK, 128), jnp.int32),
        pltpu.VMEM((ZCH,), jnp.float32),
        pltpu.VMEM_SHARED((ACC,), jnp.float32),
        pltpu.SemaphoreType.DMA,
    ],
)
def _unpool_sc(upd_hbm, mask_hbm, out_hbm, u_raw, m_raw, uv, iv, zbuf, acc,
               sem):
    core = lax.axis_index("c")
    sid = lax.axis_index("s")

    zero16 = jnp.zeros((16,), jnp.float32)

    def zinit(i, carry):
        zbuf[pl.ds(i * 16, 16)] = zero16
        return carry

    lax.fori_loop(0, ZCH // 16, zinit, 0)

    lanes = lax.iota(jnp.int32, 16)
    third = jnp.float32(1.0 / 3.0)

    def task_body(t, carry):
        task = core * TPC + t
        e0 = sid * EPT
        o0 = sid * OPT

        for z in range(OPT // ZCH):
            pltpu.sync_copy(zbuf, acc.at[pl.ds(o0 + z * ZCH, ZCH)])
        plsc.subcore_barrier()

        pltpu.sync_copy(upd_hbm.at[task, pl.ds(e0, EPT)], u_raw)
        pltpu.sync_copy(mask_hbm.at[task, pl.ds(e0, EPT)], m_raw)

        def crow(j, carry):
            m = m_raw[pl.ds(j * 16, 16)]
            u = u_raw[pl.ds(j * 16, 16)]
            t6 = lax.shift_right_logical(m, 6)
            s = (t6.astype(jnp.float32) * third).astype(jnp.int32)
            idx = s * CB + lanes
            cj = j // 8
            off = (j % 8) * 16
            iv[cj, pl.ds(off, 16)] = idx
            uv[cj, pl.ds(off, 16)] = u
            return carry

        lax.fori_loop(0, EPT // 16, crow, 0)

        def cscat(cj, carry):
            pltpu.async_copy(uv.at[cj], acc.at[iv.at[cj]], sem, add=True)
            return carry

        lax.fori_loop(0, NCHUNK, cscat, 0)

        def cdrain(cj, carry):
            pltpu.make_async_copy(uv.at[cj], acc.at[iv.at[cj]], sem).wait()
            return carry

        lax.fori_loop(0, NCHUNK, cdrain, 0)
        plsc.subcore_barrier()

        pltpu.sync_copy(acc.at[pl.ds(o0, OPT)], out_hbm.at[task, pl.ds(o0, OPT)])
        return carry

    lax.fori_loop(0, TPC, task_body, 0)


def kernel(updates, mask):
    ub = (updates.reshape(B, HW, NCB, CB).transpose(0, 2, 1, 3)
          .reshape(NTASK, HW * CB))
    mb = (mask.astype(jnp.int32).reshape(B, HW, NCB, CB).transpose(0, 2, 1, 3)
          .reshape(NTASK, HW * CB))
    out = _unpool_sc(ub, mb)
    return (out.reshape(B, NCB, oHW, CB).transpose(0, 2, 1, 3)
            .reshape(B, oH, oW, C))
